# baseline (speedup 1.0000x reference)
import jax
import jax.numpy as jnp
from jax.experimental import pallas as pl

_BS = 256


def _add_kernel(x1_ref, x2_ref, p_ref, o_ref):
    p = p_ref[...][:, None, :]
    D2 = x1_ref.shape[-1]
    o_ref[:, :, :D2] = x1_ref[...] + p[:, :, :D2]
    o_ref[:, :, D2:] = x2_ref[...] + p[:, :, D2:]


def kernel(x, pos_table):
    S, B, D = x.shape
    return pl.pallas_call(
        _add_kernel,
        grid=(S // _BS,),
        in_specs=[
            pl.BlockSpec((_BS, B, D // 2), lambda i: (i, 0, 0)),
            pl.BlockSpec((_BS, B, D // 2), lambda i: (i, 0, 1)),
            pl.BlockSpec((_BS, D), lambda i: (i, 0)),
        ],
        out_specs=pl.BlockSpec((_BS, B, D), lambda i: (i, 0, 0)),
        out_shape=jax.ShapeDtypeStruct((S, B, D), x.dtype),
    )(x, x, pos_table)
